# baseline (speedup 1.0000x reference)
import jax
import jax.numpy as jnp
from jax import lax
from jax.experimental import pallas as pl
from jax.experimental.pallas import tpu as pltpu
from jax.experimental.pallas import tpu_sc as plsc

N = 10000
E = 160000
D_IN = 48
D_HID = 128
D_OUT = 27458

NC = 2
NS = 16
NW = NC * NS
K = 128
CH = 40
NB = 5
DH = D_HID // NC
CH2 = 80
NB2 = 4
EPT2 = CH2 * K
EP = NW * CH * K
RPT = 632
NR = RPT * NS

_mesh = plsc.VectorSubcoreMesh(
    core_axis_name="c", subcore_axis_name="s", num_cores=NC, num_subcores=NS)


def _sc_aggregate(table, src_w, dst_w, zrow, gather):
    D = table.shape[1]

    def body(table_hbm, src_hbm, dst_hbm, zrow_hbm, out_hbm,
             src_l, dst_l, accum, gsem, ssem, *bufs):
        c = lax.axis_index("c")
        s = lax.axis_index("s")
        w = s * NC + c
        pltpu.sync_copy(zrow_hbm, accum.at[pl.ds(s * RPT, RPT)])
        if gather:
            pltpu.sync_copy(src_hbm.at[w], src_l)
        pltpu.sync_copy(dst_hbm.at[w], dst_l)
        if not gather:
            pltpu.sync_copy(table_hbm, bufs[0])
        plsc.subcore_barrier()

        def group(g, carry):
            base = g * 2 * NB
            if gather:
                scs = []
                for half in range(2):
                    cps = [pltpu.async_copy(
                        table_hbm.at[src_l.at[base + half * NB + b]],
                        bufs[half * NB + b], gsem) for b in range(NB)]
                    for b in range(NB):
                        cps[b].wait()
                        scs.append(pltpu.async_copy(
                            bufs[half * NB + b],
                            accum.at[dst_l.at[base + half * NB + b]], ssem,
                            add=True))
                for sc in scs:
                    sc.wait()
            else:
                scs = [pltpu.async_copy(
                    bufs[0], accum.at[dst_l.at[base + b]], ssem, add=True)
                    for b in range(2 * NB)]
                for sc in scs:
                    sc.wait()
            return carry

        lax.fori_loop(0, CH // (2 * NB), group, 0)
        plsc.subcore_barrier()
        pltpu.sync_copy(accum.at[pl.ds(s * RPT, RPT)],
                        out_hbm.at[c, pl.ds(s * RPT, RPT)])

    scratch = [
        pltpu.VMEM((CH, K), jnp.int32),
        pltpu.VMEM((CH, K), jnp.int32),
        pltpu.VMEM_SHARED((NR, D), jnp.float32),
        pltpu.SemaphoreType.DMA,
        pltpu.SemaphoreType.DMA,
    ] + [pltpu.VMEM((K, D), jnp.float32) for _ in range(2 * NB)]
    k = pl.kernel(
        body,
        out_type=jax.ShapeDtypeStruct((NC, NR, D), jnp.float32),
        mesh=_mesh, scratch_types=scratch,
        compiler_params=pltpu.CompilerParams(use_tc_tiling_on_sc=False))
    return k(table, src_w, dst_w, zrow)


def _sc_aggregate_colsplit(table2, src_q, dst_q, zrow):
    DH = table2.shape[1]

    def body(table_hbm, src_hbm, dst_hbm, zrow_hbm, out_hbm,
             src_l, dst_l, accum, gsem, ssem, *bufs):
        c = lax.axis_index("c")
        s = lax.axis_index("s")
        pltpu.sync_copy(zrow_hbm, accum.at[pl.ds(s * RPT, RPT)])
        pltpu.sync_copy(src_hbm.at[c, s], src_l)
        pltpu.sync_copy(dst_hbm.at[s], dst_l)
        plsc.subcore_barrier()

        def group(g, carry):
            base = g * 2 * NB2
            gA = [pltpu.async_copy(
                table_hbm.at[src_l.at[base + b]], bufs[b], gsem)
                for b in range(NB2)]
            sA = []
            for b in range(NB2):
                gA[b].wait()
                sA.append(pltpu.async_copy(
                    bufs[b], accum.at[dst_l.at[base + b]], ssem, add=True))
            gB = [pltpu.async_copy(
                table_hbm.at[src_l.at[base + NB2 + b]], bufs[NB2 + b], gsem)
                for b in range(NB2)]
            sB = []
            for b in range(NB2):
                gB[b].wait()
                sB.append(pltpu.async_copy(
                    bufs[NB2 + b], accum.at[dst_l.at[base + NB2 + b]], ssem,
                    add=True))
            for sc in sA + sB:
                sc.wait()
            return carry

        lax.fori_loop(0, CH2 // (2 * NB2), group, 0)
        plsc.subcore_barrier()
        pltpu.sync_copy(accum.at[pl.ds(s * RPT, RPT)],
                        out_hbm.at[c, pl.ds(s * RPT, RPT)])

    scratch = [
        pltpu.VMEM((CH2, K), jnp.int32),
        pltpu.VMEM((CH2, K), jnp.int32),
        pltpu.VMEM_SHARED((NR, DH), jnp.float32),
        pltpu.SemaphoreType.DMA,
        pltpu.SemaphoreType.DMA,
    ] + [pltpu.VMEM((K, DH), jnp.float32) for _ in range(2 * NB2)]
    k = pl.kernel(
        body,
        out_type=jax.ShapeDtypeStruct((NC, NR, DH), jnp.float32),
        mesh=_mesh, scratch_types=scratch,
        compiler_params=pltpu.CompilerParams(use_tc_tiling_on_sc=False))
    return k(table2, src_q, dst_q, zrow)


def _prep(degp, x):
    R = 2000
    grid = (N // R,)

    def body(degp_ref, x_ref, dinv_ref, xs_ref):
        deg = degp_ref[0, :, 0:1] + degp_ref[1, :, 0:1] + 1.0
        dv = lax.rsqrt(deg)
        dinv_ref[...] = dv
        xs_ref[...] = x_ref[...] * dv

    return pl.pallas_call(
        body,
        grid=grid,
        in_specs=[
            pl.BlockSpec((NC, R, 16), lambda r: (0, r, 0)),
            pl.BlockSpec((R, D_IN), lambda r: (r, 0)),
        ],
        out_specs=[
            pl.BlockSpec((R, 1), lambda r: (r, 0)),
            pl.BlockSpec((R, D_IN), lambda r: (r, 0)),
        ],
        out_shape=[
            jax.ShapeDtypeStruct((N, 1), jnp.float32),
            jax.ShapeDtypeStruct((N, D_IN), jnp.float32),
        ],
    )(degp, x)


def _layer1(pb, xs, dinv, W1, b1r):
    R = 2000
    grid = (N // R,)

    def body(pb_ref, xs_ref, dinv_ref, w_ref, b_ref, hs_ref):
        dv = dinv_ref[...]
        agg = dv * (xs_ref[...] + pb_ref[0] + pb_ref[1])
        h = jnp.dot(agg, w_ref[...], preferred_element_type=jnp.float32)
        h = dv * jnp.maximum(h + b_ref[...], 0.0)
        hs_ref[0] = h[:, :DH]
        hs_ref[1] = h[:, DH:]

    return pl.pallas_call(
        body,
        grid=grid,
        in_specs=[
            pl.BlockSpec((NC, R, D_IN), lambda r: (0, r, 0)),
            pl.BlockSpec((R, D_IN), lambda r: (r, 0)),
            pl.BlockSpec((R, 1), lambda r: (r, 0)),
            pl.BlockSpec((D_IN, D_HID), lambda r: (0, 0)),
            pl.BlockSpec((1, D_HID), lambda r: (0, 0)),
        ],
        out_specs=pl.BlockSpec((NC, R, DH), lambda r: (0, r, 0)),
        out_shape=jax.ShapeDtypeStruct((NC, N, DH), jnp.float32),
    )(pb, xs, dinv, W1, b1r)


def _layer2(pc, hs, dinv, W2p, b2p):
    R = 2000
    CB = 1536
    ncb = pl.cdiv(W2p.shape[1], CB)
    grid = (N // R, ncb)

    def body(pc_ref, hs_ref, dinv_ref, w_ref, b_ref, out_ref, acc_ref):
        @pl.when(pl.program_id(1) == 0)
        def _():
            s0 = hs_ref[0] + pc_ref[0]
            s1 = hs_ref[1] + pc_ref[1]
            agg = dinv_ref[...] * jnp.concatenate([s0, s1], axis=1)
            acc_ref[...] = agg.astype(jnp.bfloat16)

        y = jnp.dot(acc_ref[...], w_ref[...],
                    preferred_element_type=jnp.float32)
        out_ref[...] = jax.nn.sigmoid(y + b_ref[...])

    return pl.pallas_call(
        body,
        grid=grid,
        in_specs=[
            pl.BlockSpec((NC, R, DH), lambda r, c: (0, r, 0)),
            pl.BlockSpec((NC, R, DH), lambda r, c: (0, r, 0)),
            pl.BlockSpec((R, 1), lambda r, c: (r, 0)),
            pl.BlockSpec((D_HID, CB), lambda r, c: (0, c)),
            pl.BlockSpec((1, CB), lambda r, c: (0, c)),
        ],
        out_specs=pl.BlockSpec((R, CB), lambda r, c: (r, c)),
        out_shape=jax.ShapeDtypeStruct((N, D_OUT), jnp.float32),
        scratch_shapes=[pltpu.VMEM((R, D_HID), jnp.bfloat16)],
    )(pc, hs, dinv, W2p, b2p)


def kernel(x, edge_index, W1, b1, W2, b2):
    src = edge_index[0]
    dst = edge_index[1]
    trash = N + (jnp.arange(EP - E, dtype=jnp.int32) % (NR - N))
    src_w = jnp.concatenate(
        [src, jnp.zeros((EP - E,), jnp.int32)]).reshape(NW, CH, K)
    dst_w = jnp.concatenate([dst, trash]).reshape(NW, CH, K)

    ones_tab = jnp.ones((K, 16), jnp.float32)
    z16 = jnp.zeros((RPT, 16), jnp.float32)
    z48 = jnp.zeros((RPT, D_IN), jnp.float32)

    trash2 = N + (jnp.arange(NS * EPT2 - E, dtype=jnp.int32) % (NR - N))
    src_q = jnp.concatenate(
        [src, jnp.zeros((NS * EPT2 - E,), jnp.int32)]).reshape(NS, CH2, K)
    dst_q = jnp.concatenate([dst, trash2]).reshape(NS, CH2, K)
    src_q = jnp.stack([src_q, src_q + N])
    z64 = jnp.zeros((RPT, DH), jnp.float32)

    degp = _sc_aggregate(ones_tab, dst_w, dst_w, z16, gather=False)
    dinv, xs = _prep(degp, x)
    pb = _sc_aggregate(xs, src_w, dst_w, z48, gather=True)
    hs = _layer1(pb, xs, dinv, W1, b1.reshape(1, D_HID))
    pc = _sc_aggregate_colsplit(hs.reshape(NC * N, DH), src_q, dst_q, z64)

    CB = 1536
    colp = -D_OUT % CB
    W2p = jnp.pad(W2, ((0, 0), (0, colp))).astype(jnp.bfloat16)
    b2p = jnp.pad(b2, (0, colp)).reshape(1, -1)
    return _layer2(pc, hs, dinv, W2p, b2p)


# speedup vs baseline: 1.1076x; 1.1076x over previous
import jax
import jax.numpy as jnp
from jax import lax
from jax.experimental import pallas as pl
from jax.experimental.pallas import tpu as pltpu
from jax.experimental.pallas import tpu_sc as plsc

N = 10000
E = 160000
D_IN = 48
D_HID = 128
D_OUT = 27458

NC = 2
NS = 16
NW = NC * NS
K = 128
CH = 40
NB = 5
DH = D_HID // NC
CH2 = 80
NB2 = 4
EPT2 = CH2 * K
EP = NW * CH * K
RPT = 632
NR = RPT * NS

_mesh = plsc.VectorSubcoreMesh(
    core_axis_name="c", subcore_axis_name="s", num_cores=NC, num_subcores=NS)


def _sc_aggregate(table, src_w, dst_w, zrow, gather):
    D = table.shape[1]

    def body(table_hbm, src_hbm, dst_hbm, zrow_hbm, out_hbm,
             src_l, dst_l, accum, gsem, ssem, *bufs):
        c = lax.axis_index("c")
        s = lax.axis_index("s")
        w = s * NC + c
        pltpu.sync_copy(zrow_hbm, accum.at[pl.ds(s * RPT, RPT)])
        if gather:
            pltpu.sync_copy(src_hbm.at[w], src_l)
        pltpu.sync_copy(dst_hbm.at[w], dst_l)
        if not gather:
            pltpu.sync_copy(table_hbm, bufs[0])
        plsc.subcore_barrier()

        def group(g, carry):
            base = g * 2 * NB
            if gather:
                scs = []
                for half in range(2):
                    cps = [pltpu.async_copy(
                        table_hbm.at[src_l.at[base + half * NB + b]],
                        bufs[half * NB + b], gsem) for b in range(NB)]
                    for b in range(NB):
                        cps[b].wait()
                        scs.append(pltpu.async_copy(
                            bufs[half * NB + b],
                            accum.at[dst_l.at[base + half * NB + b]], ssem,
                            add=True))
                for sc in scs:
                    sc.wait()
            else:
                scs = [pltpu.async_copy(
                    bufs[0], accum.at[dst_l.at[base + b]], ssem, add=True)
                    for b in range(2 * NB)]
                for sc in scs:
                    sc.wait()
            return carry

        lax.fori_loop(0, CH // (2 * NB), group, 0)
        plsc.subcore_barrier()
        pltpu.sync_copy(accum.at[pl.ds(s * RPT, RPT)],
                        out_hbm.at[c, pl.ds(s * RPT, RPT)])

    scratch = [
        pltpu.VMEM((CH, K), jnp.int32),
        pltpu.VMEM((CH, K), jnp.int32),
        pltpu.VMEM_SHARED((NR, D), jnp.float32),
        pltpu.SemaphoreType.DMA,
        pltpu.SemaphoreType.DMA,
    ] + [pltpu.VMEM((K, D), jnp.float32) for _ in range(2 * NB)]
    k = pl.kernel(
        body,
        out_type=jax.ShapeDtypeStruct((NC, NR, D), jnp.float32),
        mesh=_mesh, scratch_types=scratch,
        compiler_params=pltpu.CompilerParams(use_tc_tiling_on_sc=False))
    return k(table, src_w, dst_w, zrow)


def _sc_aggregate_colsplit(table2, src_q, dst_q, zrow):
    DH = table2.shape[1]

    def body(table_hbm, src_hbm, dst_hbm, zrow_hbm, out_hbm,
             src_l, dst_l, accum, gsem, ssem, *bufs):
        c = lax.axis_index("c")
        s = lax.axis_index("s")
        pltpu.sync_copy(zrow_hbm, accum.at[pl.ds(s * RPT, RPT)])
        pltpu.sync_copy(src_hbm.at[c, s], src_l)
        pltpu.sync_copy(dst_hbm.at[s], dst_l)
        plsc.subcore_barrier()

        def group(g, carry):
            base = g * 2 * NB2
            gA = [pltpu.async_copy(
                table_hbm.at[src_l.at[base + b]], bufs[b], gsem)
                for b in range(NB2)]
            sA = []
            for b in range(NB2):
                gA[b].wait()
                sA.append(pltpu.async_copy(
                    bufs[b], accum.at[dst_l.at[base + b]], ssem, add=True))
            gB = [pltpu.async_copy(
                table_hbm.at[src_l.at[base + NB2 + b]], bufs[NB2 + b], gsem)
                for b in range(NB2)]
            sB = []
            for b in range(NB2):
                gB[b].wait()
                sB.append(pltpu.async_copy(
                    bufs[NB2 + b], accum.at[dst_l.at[base + NB2 + b]], ssem,
                    add=True))
            for sc in sA + sB:
                sc.wait()
            return carry

        lax.fori_loop(0, CH2 // (2 * NB2), group, 0)
        plsc.subcore_barrier()
        pltpu.sync_copy(accum.at[pl.ds(s * RPT, RPT)],
                        out_hbm.at[c, pl.ds(s * RPT, RPT)])

    scratch = [
        pltpu.VMEM((CH2, K), jnp.int32),
        pltpu.VMEM((CH2, K), jnp.int32),
        pltpu.VMEM_SHARED((NR, DH), jnp.bfloat16),
        pltpu.SemaphoreType.DMA,
        pltpu.SemaphoreType.DMA,
    ] + [pltpu.VMEM((K, DH), jnp.bfloat16) for _ in range(2 * NB2)]
    k = pl.kernel(
        body,
        out_type=jax.ShapeDtypeStruct((NC, NR, DH), jnp.bfloat16),
        mesh=_mesh, scratch_types=scratch,
        compiler_params=pltpu.CompilerParams(use_tc_tiling_on_sc=False))
    return k(table2, src_q, dst_q, zrow)


def _prep(degp, x):
    R = 2000
    grid = (N // R,)

    def body(degp_ref, x_ref, dinv_ref, xs_ref):
        deg = degp_ref[0, :, 0:1] + degp_ref[1, :, 0:1] + 1.0
        dv = lax.rsqrt(deg)
        dinv_ref[...] = dv
        xs_ref[...] = x_ref[...] * dv

    return pl.pallas_call(
        body,
        grid=grid,
        in_specs=[
            pl.BlockSpec((NC, R, 16), lambda r: (0, r, 0)),
            pl.BlockSpec((R, D_IN), lambda r: (r, 0)),
        ],
        out_specs=[
            pl.BlockSpec((R, 1), lambda r: (r, 0)),
            pl.BlockSpec((R, D_IN), lambda r: (r, 0)),
        ],
        out_shape=[
            jax.ShapeDtypeStruct((N, 1), jnp.float32),
            jax.ShapeDtypeStruct((N, D_IN), jnp.float32),
        ],
    )(degp, x)


def _layer1(pb, xs, dinv, W1, b1r):
    R = 2000
    grid = (N // R,)

    def body(pb_ref, xs_ref, dinv_ref, w_ref, b_ref, hs_ref):
        dv = dinv_ref[...]
        agg = dv * (xs_ref[...] + pb_ref[0] + pb_ref[1])
        h = jnp.dot(agg, w_ref[...], preferred_element_type=jnp.float32)
        h = (dv * jnp.maximum(h + b_ref[...], 0.0)).astype(jnp.bfloat16)
        hs_ref[0] = h[:, :DH]
        hs_ref[1] = h[:, DH:]

    return pl.pallas_call(
        body,
        grid=grid,
        in_specs=[
            pl.BlockSpec((NC, R, D_IN), lambda r: (0, r, 0)),
            pl.BlockSpec((R, D_IN), lambda r: (r, 0)),
            pl.BlockSpec((R, 1), lambda r: (r, 0)),
            pl.BlockSpec((D_IN, D_HID), lambda r: (0, 0)),
            pl.BlockSpec((1, D_HID), lambda r: (0, 0)),
        ],
        out_specs=pl.BlockSpec((NC, R, DH), lambda r: (0, r, 0)),
        out_shape=jax.ShapeDtypeStruct((NC, N, DH), jnp.bfloat16),
    )(pb, xs, dinv, W1, b1r)


def _layer2(pc, hs, dinv, W2p, b2p):
    R = 2000
    CB = 1536
    ncb = pl.cdiv(W2p.shape[1], CB)
    grid = (N // R, ncb)

    def body(pc_ref, hs_ref, dinv_ref, w_ref, b_ref, out_ref, acc_ref):
        @pl.when(pl.program_id(1) == 0)
        def _():
            s0 = (hs_ref[0] + pc_ref[0]).astype(jnp.float32)
            s1 = (hs_ref[1] + pc_ref[1]).astype(jnp.float32)
            acc_ref[...] = dinv_ref[...] * jnp.concatenate([s0, s1], axis=1)

        y = jnp.dot(acc_ref[...], w_ref[...],
                    preferred_element_type=jnp.float32)
        out_ref[...] = jax.nn.sigmoid(y + b_ref[...])

    return pl.pallas_call(
        body,
        grid=grid,
        in_specs=[
            pl.BlockSpec((NC, R, DH), lambda r, c: (0, r, 0)),
            pl.BlockSpec((NC, R, DH), lambda r, c: (0, r, 0)),
            pl.BlockSpec((R, 1), lambda r, c: (r, 0)),
            pl.BlockSpec((D_HID, CB), lambda r, c: (0, c)),
            pl.BlockSpec((1, CB), lambda r, c: (0, c)),
        ],
        out_specs=pl.BlockSpec((R, CB), lambda r, c: (r, c)),
        out_shape=jax.ShapeDtypeStruct((N, D_OUT), jnp.float32),
        scratch_shapes=[pltpu.VMEM((R, D_HID), jnp.float32)],
    )(pc, hs, dinv, W2p, b2p)


def kernel(x, edge_index, W1, b1, W2, b2):
    src = edge_index[0]
    dst = edge_index[1]
    trash = N + (jnp.arange(EP - E, dtype=jnp.int32) % (NR - N))
    src_w = jnp.concatenate(
        [src, jnp.zeros((EP - E,), jnp.int32)]).reshape(NW, CH, K)
    dst_w = jnp.concatenate([dst, trash]).reshape(NW, CH, K)

    ones_tab = jnp.ones((K, 16), jnp.float32)
    z16 = jnp.zeros((RPT, 16), jnp.float32)
    z48 = jnp.zeros((RPT, D_IN), jnp.float32)

    trash2 = N + (jnp.arange(NS * EPT2 - E, dtype=jnp.int32) % (NR - N))
    src_q = jnp.concatenate(
        [src, jnp.zeros((NS * EPT2 - E,), jnp.int32)]).reshape(NS, CH2, K)
    dst_q = jnp.concatenate([dst, trash2]).reshape(NS, CH2, K)
    src_q = jnp.stack([src_q, src_q + N])
    z64 = jnp.zeros((RPT, DH), jnp.bfloat16)

    degp = _sc_aggregate(ones_tab, dst_w, dst_w, z16, gather=False)
    dinv, xs = _prep(degp, x)
    pb = _sc_aggregate(xs, src_w, dst_w, z48, gather=True)
    hs = _layer1(pb, xs, dinv, W1, b1.reshape(1, D_HID))
    pc = _sc_aggregate_colsplit(hs.reshape(NC * N, DH), src_q, dst_q, z64)

    CB = 1536
    colp = -D_OUT % CB
    W2p = jnp.pad(W2, ((0, 0), (0, colp)))
    b2p = jnp.pad(b2, (0, colp)).reshape(1, -1)
    return _layer2(pc, hs, dinv, W2p, b2p)


# speedup vs baseline: 1.1675x; 1.0541x over previous
import jax
import jax.numpy as jnp
from jax import lax
from jax.experimental import pallas as pl
from jax.experimental.pallas import tpu as pltpu
from jax.experimental.pallas import tpu_sc as plsc

N = 10000
E = 160000
D_IN = 48
D_HID = 128
D_OUT = 27458

NC = 2
NS = 16
NW = NC * NS
K = 128
CH = 40
NB = 5
DH = D_HID // NC
CH2 = 80
NB2 = 4
EPT2 = CH2 * K
EP = NW * CH * K
RPT = 632
NR = RPT * NS

_mesh = plsc.VectorSubcoreMesh(
    core_axis_name="c", subcore_axis_name="s", num_cores=NC, num_subcores=NS)


def _sc_aggregate(table, src_w, dst_w, zrow, gather):
    D = table.shape[1]
    dt = table.dtype

    def body(table_hbm, src_hbm, dst_hbm, zrow_hbm, out_hbm,
             src_l, dst_l, accum, gsem, ssem, *bufs):
        c = lax.axis_index("c")
        s = lax.axis_index("s")
        w = s * NC + c
        pltpu.sync_copy(zrow_hbm, accum.at[pl.ds(s * RPT, RPT)])
        if gather:
            pltpu.sync_copy(src_hbm.at[w], src_l)
        pltpu.sync_copy(dst_hbm.at[w], dst_l)
        if not gather:
            pltpu.sync_copy(table_hbm, bufs[0])
        plsc.subcore_barrier()

        def group(g, carry):
            base = g * 2 * NB
            if gather:
                scs = []
                for half in range(2):
                    cps = [pltpu.async_copy(
                        table_hbm.at[src_l.at[base + half * NB + b]],
                        bufs[half * NB + b], gsem) for b in range(NB)]
                    for b in range(NB):
                        cps[b].wait()
                        scs.append(pltpu.async_copy(
                            bufs[half * NB + b],
                            accum.at[dst_l.at[base + half * NB + b]], ssem,
                            add=True))
                for sc in scs:
                    sc.wait()
            else:
                scs = [pltpu.async_copy(
                    bufs[0], accum.at[dst_l.at[base + b]], ssem, add=True)
                    for b in range(2 * NB)]
                for sc in scs:
                    sc.wait()
            return carry

        lax.fori_loop(0, CH // (2 * NB), group, 0)
        plsc.subcore_barrier()
        pltpu.sync_copy(accum.at[pl.ds(s * RPT, RPT)],
                        out_hbm.at[c, pl.ds(s * RPT, RPT)])

    scratch = [
        pltpu.VMEM((CH, K), jnp.int32),
        pltpu.VMEM((CH, K), jnp.int32),
        pltpu.VMEM_SHARED((NR, D), dt),
        pltpu.SemaphoreType.DMA,
        pltpu.SemaphoreType.DMA,
    ] + [pltpu.VMEM((K, D), dt) for _ in range(2 * NB)]
    k = pl.kernel(
        body,
        out_type=jax.ShapeDtypeStruct((NC, NR, D), dt),
        mesh=_mesh, scratch_types=scratch,
        compiler_params=pltpu.CompilerParams(use_tc_tiling_on_sc=False))
    return k(table, src_w, dst_w, zrow)


def _sc_aggregate_colsplit(table2, src_q, dst_q, zrow):
    DH = table2.shape[1]

    def body(table_hbm, src_hbm, dst_hbm, zrow_hbm, out_hbm,
             src_l, dst_l, accum, gsem, ssem, *bufs):
        c = lax.axis_index("c")
        s = lax.axis_index("s")
        pltpu.sync_copy(zrow_hbm, accum.at[pl.ds(s * RPT, RPT)])
        pltpu.sync_copy(src_hbm.at[c, s], src_l)
        pltpu.sync_copy(dst_hbm.at[s], dst_l)
        plsc.subcore_barrier()

        def group(g, carry):
            base = g * 2 * NB2
            gA = [pltpu.async_copy(
                table_hbm.at[src_l.at[base + b]], bufs[b], gsem)
                for b in range(NB2)]
            sA = []
            for b in range(NB2):
                gA[b].wait()
                sA.append(pltpu.async_copy(
                    bufs[b], accum.at[dst_l.at[base + b]], ssem, add=True))
            gB = [pltpu.async_copy(
                table_hbm.at[src_l.at[base + NB2 + b]], bufs[NB2 + b], gsem)
                for b in range(NB2)]
            sB = []
            for b in range(NB2):
                gB[b].wait()
                sB.append(pltpu.async_copy(
                    bufs[NB2 + b], accum.at[dst_l.at[base + NB2 + b]], ssem,
                    add=True))
            for sc in sA + sB:
                sc.wait()
            return carry

        lax.fori_loop(0, CH2 // (2 * NB2), group, 0)
        plsc.subcore_barrier()
        pltpu.sync_copy(accum.at[pl.ds(s * RPT, RPT)],
                        out_hbm.at[c, pl.ds(s * RPT, RPT)])

    scratch = [
        pltpu.VMEM((CH2, K), jnp.int32),
        pltpu.VMEM((CH2, K), jnp.int32),
        pltpu.VMEM_SHARED((NR, DH), jnp.bfloat16),
        pltpu.SemaphoreType.DMA,
        pltpu.SemaphoreType.DMA,
    ] + [pltpu.VMEM((K, DH), jnp.bfloat16) for _ in range(2 * NB2)]
    k = pl.kernel(
        body,
        out_type=jax.ShapeDtypeStruct((NC, NR, DH), jnp.bfloat16),
        mesh=_mesh, scratch_types=scratch,
        compiler_params=pltpu.CompilerParams(use_tc_tiling_on_sc=False))
    return k(table2, src_q, dst_q, zrow)


def _prep(degp, x):
    R = 2000
    grid = (N // R,)

    def body(degp_ref, x_ref, dinv_ref, xs_ref):
        deg = degp_ref[0, :, 0:1] + degp_ref[1, :, 0:1] + 1.0
        dv = lax.rsqrt(deg)
        dinv_ref[...] = dv
        xs_ref[...] = x_ref[...] * dv

    return pl.pallas_call(
        body,
        grid=grid,
        in_specs=[
            pl.BlockSpec((NC, R, 16), lambda r: (0, r, 0)),
            pl.BlockSpec((R, D_IN), lambda r: (r, 0)),
        ],
        out_specs=[
            pl.BlockSpec((R, 1), lambda r: (r, 0)),
            pl.BlockSpec((R, D_IN), lambda r: (r, 0)),
        ],
        out_shape=[
            jax.ShapeDtypeStruct((N, 1), jnp.float32),
            jax.ShapeDtypeStruct((N, D_IN), jnp.float32),
        ],
    )(degp, x)


def _layer1(pb, xs, dinv, W1, b1r):
    R = 2000
    grid = (N // R,)

    def body(pb_ref, xs_ref, dinv_ref, w_ref, b_ref, hs_ref):
        dv = dinv_ref[...]
        agg = dv * (xs_ref[...]
                    + (pb_ref[0] + pb_ref[1]).astype(jnp.float32))
        h = jnp.dot(agg, w_ref[...], preferred_element_type=jnp.float32)
        h = (dv * jnp.maximum(h + b_ref[...], 0.0)).astype(jnp.bfloat16)
        hs_ref[0] = h[:, :DH]
        hs_ref[1] = h[:, DH:]

    return pl.pallas_call(
        body,
        grid=grid,
        in_specs=[
            pl.BlockSpec((NC, R, D_IN), lambda r: (0, r, 0)),
            pl.BlockSpec((R, D_IN), lambda r: (r, 0)),
            pl.BlockSpec((R, 1), lambda r: (r, 0)),
            pl.BlockSpec((D_IN, D_HID), lambda r: (0, 0)),
            pl.BlockSpec((1, D_HID), lambda r: (0, 0)),
        ],
        out_specs=pl.BlockSpec((NC, R, DH), lambda r: (0, r, 0)),
        out_shape=jax.ShapeDtypeStruct((NC, N, DH), jnp.bfloat16),
    )(pb, xs, dinv, W1, b1r)


def _layer2(pc, hs, dinv, W2p, b2p):
    R = 2000
    CB = 1536
    ncb = pl.cdiv(W2p.shape[1], CB)
    grid = (N // R, ncb)

    def body(pc_ref, hs_ref, dinv_ref, w_ref, b_ref, out_ref, acc_ref):
        @pl.when(pl.program_id(1) == 0)
        def _():
            s0 = (hs_ref[0] + pc_ref[0]).astype(jnp.float32)
            s1 = (hs_ref[1] + pc_ref[1]).astype(jnp.float32)
            acc_ref[...] = dinv_ref[...] * jnp.concatenate([s0, s1], axis=1)

        y = jnp.dot(acc_ref[...], w_ref[...],
                    preferred_element_type=jnp.float32)
        out_ref[...] = jax.nn.sigmoid(y + b_ref[...])

    return pl.pallas_call(
        body,
        grid=grid,
        in_specs=[
            pl.BlockSpec((NC, R, DH), lambda r, c: (0, r, 0)),
            pl.BlockSpec((NC, R, DH), lambda r, c: (0, r, 0)),
            pl.BlockSpec((R, 1), lambda r, c: (r, 0)),
            pl.BlockSpec((D_HID, CB), lambda r, c: (0, c)),
            pl.BlockSpec((1, CB), lambda r, c: (0, c)),
        ],
        out_specs=pl.BlockSpec((R, CB), lambda r, c: (r, c)),
        out_shape=jax.ShapeDtypeStruct((N, D_OUT), jnp.float32),
        scratch_shapes=[pltpu.VMEM((R, D_HID), jnp.float32)],
    )(pc, hs, dinv, W2p, b2p)


def kernel(x, edge_index, W1, b1, W2, b2):
    src = edge_index[0]
    dst = edge_index[1]
    trash = N + (jnp.arange(EP - E, dtype=jnp.int32) % (NR - N))
    src_w = jnp.concatenate(
        [src, jnp.zeros((EP - E,), jnp.int32)]).reshape(NW, CH, K)
    dst_w = jnp.concatenate([dst, trash]).reshape(NW, CH, K)

    ones_tab = jnp.ones((K, 16), jnp.float32)
    z16 = jnp.zeros((RPT, 16), jnp.float32)
    z48 = jnp.zeros((RPT, D_IN), jnp.bfloat16)

    trash2 = N + (jnp.arange(NS * EPT2 - E, dtype=jnp.int32) % (NR - N))
    src_q = jnp.concatenate(
        [src, jnp.zeros((NS * EPT2 - E,), jnp.int32)]).reshape(NS, CH2, K)
    dst_q = jnp.concatenate([dst, trash2]).reshape(NS, CH2, K)
    src_q = jnp.stack([src_q, src_q + N])
    z64 = jnp.zeros((RPT, DH), jnp.bfloat16)

    degp = _sc_aggregate(ones_tab, dst_w, dst_w, z16, gather=False)
    dinv, xs = _prep(degp, x)
    pb = _sc_aggregate(xs.astype(jnp.bfloat16), src_w, dst_w, z48,
                       gather=True)
    hs = _layer1(pb, xs, dinv, W1, b1.reshape(1, D_HID))
    pc = _sc_aggregate_colsplit(hs.reshape(NC * N, DH), src_q, dst_q, z64)

    CB = 1536
    colp = -D_OUT % CB
    W2p = jnp.pad(W2, ((0, 0), (0, colp)))
    b2p = jnp.pad(b2, (0, colp)).reshape(1, -1)
    return _layer2(pc, hs, dinv, W2p, b2p)
